# baseline (speedup 1.0000x reference)
import dataclasses
import functools

import jax
import jax.numpy as jnp
from jax import lax
from jax.experimental import pallas as pl
from jax.experimental.pallas import tpu as pltpu
from jax.experimental.pallas import tpu_sc as plsc

NUM_DIM = 3
N_FEATURES = 2
LOG2_HASHMAP_SIZE = 19
MAX_PARAMS = 2 ** LOG2_HASHMAP_SIZE
RESOLUTIONS = [16, 23, 32, 46, 64, 92, 128, 184, 256, 368, 512, 736]
N_LEVELS = len(RESOLUTIONS)
OUT_F = N_LEVELS * N_FEATURES

P1 = 2654435761 - 2 ** 32
P2 = 805459861
HASH_MASK = MAX_PARAMS - 1


def _make_offsets():
    offs = [0]
    off = 0
    for r in RESOLUTIONS:
        p = min(MAX_PARAMS, r ** NUM_DIM)
        p = int(-(-p // 8) * 8)
        off += p
        offs.append(off)
    return offs

_OFFSETS = _make_offsets()

_LEVEL_PARAMS = []
for _l, _r in enumerate(RESOLUTIONS):
    _hsize = _OFFSETS[_l + 1] - _OFFSETS[_l]
    _LEVEL_PARAMS.append((float(_r - 1), _OFFSETS[_l], _r ** NUM_DIM > _hsize,
                          _r, _r * _r))

N_POINTS = 262144
NW = 32
CHUNK = 128
PPW = N_POINTS // NW
N_CHUNKS = PPW // CHUNK


def _encoder_body(x_hbm, y_hbm, z_hbm, emb_hbm, o_hbm,
                  xv, yv, zv, idx_r, sub_r, w_r, rows_r, out_r, sem):
    wid = lax.axis_index("s") * 2 + lax.axis_index("c")
    iota = lax.iota(jnp.int32, 16)

    @pl.loop(0, N_CHUNKS)
    def _chunk(t):
        base = wid * PPW + t * CHUNK
        pltpu.sync_copy(x_hbm.at[pl.ds(base, CHUNK)], xv)
        pltpu.sync_copy(y_hbm.at[pl.ds(base, CHUNK)], yv)
        pltpu.sync_copy(z_hbm.at[pl.ds(base, CHUNK)], zv)

        for L in range(N_LEVELS):
            scale, off, is_hash, sy, sz = _LEVEL_PARAMS[L]

            @pl.loop(0, CHUNK // 16)
            def _idx_pass(i):
                s = pl.ds(i * 16, 16)
                px = xv[s] * scale
                py = yv[s] * scale
                pz = zv[s] * scale
                bx = px.astype(jnp.int32)
                by = py.astype(jnp.int32)
                bz = pz.astype(jnp.int32)
                fx = px - bx.astype(jnp.float32)
                fy = py - by.astype(jnp.float32)
                fz = pz - bz.astype(jnp.float32)
                if is_hash:
                    hx = (bx, bx + 1)
                    hy0 = by * P1
                    hy = (hy0, hy0 + P1)
                    hz0 = bz * P2
                    hz = (hz0, hz0 + P2)
                else:
                    hx = (bx, bx + 1)
                    hy0 = by * sy
                    hy = (hy0, hy0 + sy)
                    hz0 = bz * sz + off
                    hz = (hz0, hz0 + sz)
                wx = (1.0 - fx, fx)
                wy = (1.0 - fy, fy)
                wz = (1.0 - fz, fz)
                for c in range(8):
                    cx, cy, cz = c & 1, (c >> 1) & 1, (c >> 2) & 1
                    if is_hash:
                        ci = ((hx[cx] ^ hy[cy] ^ hz[cz]) & HASH_MASK) + off
                    else:
                        ci = hx[cx] + hy[cy] + hz[cz]
                    idx_r[c, s] = lax.shift_right_logical(ci, 2)
                    sub_r[c, s] = lax.shift_left((ci & 3), 1)
                    w_r[c, s] = wx[cx] * (wy[cy] * wz[cz])

            copies = [
                pltpu.async_copy(emb_hbm.at[idx_r.at[c]], rows_r.at[c], sem)
                for c in range(8)
            ]
            for cp_ in copies:
                cp_.wait()

            @pl.loop(0, CHUNK // 16)
            def _acc_pass(i):
                s = pl.ds(i * 16, 16)
                pt = i * 16 + iota
                acc0 = jnp.zeros((16,), jnp.float32)
                acc1 = jnp.zeros((16,), jnp.float32)
                for c in range(8):
                    cvec = jnp.full((16,), c, jnp.int32)
                    w = w_r[c, s]
                    sub = sub_r[c, s]
                    f0 = plsc.load_gather(rows_r, [cvec, pt, sub])
                    f1 = plsc.load_gather(rows_r, [cvec, pt, sub + 1])
                    acc0 = acc0 + w * f0
                    acc1 = acc1 + w * f1
                oidx = pt * OUT_F + (2 * L)
                plsc.store_scatter(out_r, [oidx], acc0)
                plsc.store_scatter(out_r, [oidx + 1], acc1)

        pltpu.sync_copy(out_r, o_hbm.at[pl.ds(base * OUT_F, CHUNK * OUT_F)])


@jax.jit
def kernel(inputs, embeddings):
    inputs = inputs.reshape(-1, NUM_DIM)
    n = inputs.shape[0]
    x = inputs[:, 0]
    y = inputs[:, 1]
    z = inputs[:, 2]

    cp = pltpu.CompilerParams()
    for _f, _v in (("needs_layout_passes", False),
                   ("use_tc_tiling_on_sc", False)):
        if _f in pltpu.CompilerParams.__dataclass_fields__:
            cp = dataclasses.replace(cp, **{_f: _v})

    mesh = plsc.VectorSubcoreMesh(core_axis_name="c", subcore_axis_name="s")
    run = pl.kernel(
        _encoder_body,
        out_type=jax.ShapeDtypeStruct((n * OUT_F,), jnp.float32),
        mesh=mesh,
        compiler_params=cp,
        scratch_types=[
            pltpu.VMEM((CHUNK,), jnp.float32),
            pltpu.VMEM((CHUNK,), jnp.float32),
            pltpu.VMEM((CHUNK,), jnp.float32),
            pltpu.VMEM((8, CHUNK), jnp.int32),
            pltpu.VMEM((8, CHUNK), jnp.int32),
            pltpu.VMEM((8, CHUNK), jnp.float32),
            pltpu.VMEM((8, CHUNK, 8), jnp.float32),
            pltpu.VMEM((CHUNK * OUT_F,), jnp.float32),
            pltpu.SemaphoreType.DMA,
        ],
    )
    emb_quads = embeddings.reshape(-1, 8)
    out = run(x, y, z, emb_quads)
    return out.reshape(n, OUT_F)


# speedup vs baseline: 3.9904x; 3.9904x over previous
import dataclasses
import functools

import jax
import jax.numpy as jnp
from jax import lax
from jax.experimental import pallas as pl
from jax.experimental.pallas import tpu as pltpu
from jax.experimental.pallas import tpu_sc as plsc

NUM_DIM = 3
N_FEATURES = 2
LOG2_HASHMAP_SIZE = 19
MAX_PARAMS = 2 ** LOG2_HASHMAP_SIZE
RESOLUTIONS = [16, 23, 32, 46, 64, 92, 128, 184, 256, 368, 512, 736]
N_LEVELS = len(RESOLUTIONS)
OUT_F = N_LEVELS * N_FEATURES

P1 = 2654435761 - 2 ** 32
P2 = 805459861
HASH_MASK = MAX_PARAMS - 1


def _make_offsets():
    offs = [0]
    off = 0
    for r in RESOLUTIONS:
        p = min(MAX_PARAMS, r ** NUM_DIM)
        p = int(-(-p // 8) * 8)
        off += p
        offs.append(off)
    return offs

_OFFSETS = _make_offsets()

_LEVEL_PARAMS = []
for _l, _r in enumerate(RESOLUTIONS):
    _hsize = _OFFSETS[_l + 1] - _OFFSETS[_l]
    _LEVEL_PARAMS.append((float(_r - 1), _OFFSETS[_l], _r ** NUM_DIM > _hsize,
                          _r, _r * _r))

N_POINTS = 262144
NW = 32
CHUNK = 128
PPW = N_POINTS // NW
N_CHUNKS = PPW // CHUNK


N_STAGED = 3
SPMEM_LEVELS = ()
DEPTH = 4


def _encoder_body(x_hbm, y_hbm, z_hbm, e0_hbm, e1_hbm, o_hbm,
                  xv, yv, zv, idx_r, w_r, rows0_r, rows1_r,
                  out_r, t0f0, t0f1, t1f0, t1f1, t2f0, t2f1,
                  sem):
    sid = lax.axis_index("s")
    wid = sid * 2 + lax.axis_index("c")
    iota = lax.iota(jnp.int32, 16)
    staged = [(t0f0, t0f1), (t1f0, t1f1), (t2f0, t2f1)]
    spmem = {}

    stage_cp = []
    for L in range(N_STAGED):
        off = _OFFSETS[L]
        hsize = _OFFSETS[L + 1] - _OFFSETS[L]
        stage_cp.append(pltpu.async_copy(
            e0_hbm.at[pl.ds(off, hsize)], staged[L][0], sem))
        stage_cp.append(pltpu.async_copy(
            e1_hbm.at[pl.ds(off, hsize)], staged[L][1], sem))
    for cp_ in stage_cp:
        cp_.wait()

    def _idx_pass(L, p):
        scale, off, is_hash, sy, sz = _LEVEL_PARAMS[L]
        if L in SPMEM_LEVELS:
            off = 0

        @pl.loop(0, CHUNK // 16)
        def _pass(i):
            s = pl.ds(i * 16, 16)
            px = xv[s] * scale
            py = yv[s] * scale
            pz = zv[s] * scale
            bx = px.astype(jnp.int32)
            by = py.astype(jnp.int32)
            bz = pz.astype(jnp.int32)
            fx = px - bx.astype(jnp.float32)
            fy = py - by.astype(jnp.float32)
            fz = pz - bz.astype(jnp.float32)
            if is_hash:
                hx = (bx, bx + 1)
                hy0 = by * P1
                hy = (hy0, hy0 + P1)
                hz0 = bz * P2
                hz = (hz0, hz0 + P2)
            else:
                hx = (bx, bx + 1)
                hy0 = by * sy
                hy = (hy0, hy0 + sy)
                hz0 = bz * sz + off
                hz = (hz0, hz0 + sz)
            wx = (1.0 - fx, fx)
            wy = (1.0 - fy, fy)
            wz = (1.0 - fz, fz)
            for c in range(8):
                cx, cy, cz = c & 1, (c >> 1) & 1, (c >> 2) & 1
                if is_hash:
                    ci = (hx[cx] ^ hy[cy] ^ hz[cz]) & HASH_MASK
                    if off:
                        ci = ci + off
                else:
                    ci = hx[cx] + hy[cy] + hz[cz]
                idx_r[p, c, s] = ci
                w_r[p, c, s] = wx[cx] * (wy[cy] * wz[cz])

    def _issue(L, p):
        src0, src1 = spmem.get(L, (e0_hbm, e1_hbm))
        return [
            pltpu.async_copy(src0.at[idx_r.at[p, c]], rows0_r.at[p, c], sem)
            for c in range(8)
        ] + [
            pltpu.async_copy(src1.at[idx_r.at[p, c]], rows1_r.at[p, c], sem)
            for c in range(8)
        ]

    def _fused_level(L):
        scale, _off, _is_hash, sy, sz = _LEVEL_PARAMS[L]
        tf0, tf1 = staged[L]

        @pl.loop(0, CHUNK // 16)
        def _pass(i):
            s = pl.ds(i * 16, 16)
            px = xv[s] * scale
            py = yv[s] * scale
            pz = zv[s] * scale
            bx = px.astype(jnp.int32)
            by = py.astype(jnp.int32)
            bz = pz.astype(jnp.int32)
            fx = px - bx.astype(jnp.float32)
            fy = py - by.astype(jnp.float32)
            fz = pz - bz.astype(jnp.float32)
            hx = (bx, bx + 1)
            hy0 = by * sy
            hy = (hy0, hy0 + sy)
            hz0 = bz * sz
            hz = (hz0, hz0 + sz)
            wx = (1.0 - fx, fx)
            wy = (1.0 - fy, fy)
            wz = (1.0 - fz, fz)
            acc0 = jnp.zeros((16,), jnp.float32)
            acc1 = jnp.zeros((16,), jnp.float32)
            for c in range(8):
                cx, cy, cz = c & 1, (c >> 1) & 1, (c >> 2) & 1
                ci = hx[cx] + hy[cy] + hz[cz]
                w = wx[cx] * (wy[cy] * wz[cz])
                acc0 = acc0 + w * plsc.load_gather(tf0, [ci])
                acc1 = acc1 + w * plsc.load_gather(tf1, [ci])
            pt = i * 16 + iota
            oidx = pt * OUT_F + (2 * L)
            plsc.store_scatter(out_r, [oidx], acc0)
            plsc.store_scatter(out_r, [oidx + 1], acc1)

    def _acc_pass(L, p):
        @pl.loop(0, CHUNK // 16)
        def _pass(i):
            s = pl.ds(i * 16, 16)
            pt = i * 16 + iota
            acc0 = jnp.zeros((16,), jnp.float32)
            acc1 = jnp.zeros((16,), jnp.float32)
            for c in range(8):
                w = w_r[p, c, s]
                acc0 = acc0 + w * rows0_r[p, c, s]
                acc1 = acc1 + w * rows1_r[p, c, s]
            oidx = pt * OUT_F + (2 * L)
            plsc.store_scatter(out_r, [oidx], acc0)
            plsc.store_scatter(out_r, [oidx + 1], acc1)

    @pl.loop(0, N_CHUNKS)
    def _chunk(t):
        base = wid * PPW + t * CHUNK
        pltpu.sync_copy(x_hbm.at[pl.ds(base, CHUNK)], xv)
        pltpu.sync_copy(y_hbm.at[pl.ds(base, CHUNK)], yv)
        pltpu.sync_copy(z_hbm.at[pl.ds(base, CHUNK)], zv)

        copies = {}
        for L in range(N_STAGED, min(N_STAGED + DEPTH, N_LEVELS)):
            _idx_pass(L, L % DEPTH)
            copies[L] = _issue(L, L % DEPTH)
        for L in range(N_STAGED):
            _fused_level(L)
        for L in range(N_STAGED, N_LEVELS):
            for cp_ in copies.pop(L):
                cp_.wait()
            _acc_pass(L, L % DEPTH)
            if L + DEPTH < N_LEVELS:
                _idx_pass(L + DEPTH, (L + DEPTH) % DEPTH)
                copies[L + DEPTH] = _issue(L + DEPTH, (L + DEPTH) % DEPTH)

        pltpu.sync_copy(out_r, o_hbm.at[pl.ds(base * OUT_F, CHUNK * OUT_F)])


@jax.jit
def kernel(inputs, embeddings):
    inputs = inputs.reshape(-1, NUM_DIM)
    n = inputs.shape[0]
    x = inputs[:, 0]
    y = inputs[:, 1]
    z = inputs[:, 2]

    cp = pltpu.CompilerParams()
    for _f, _v in (("needs_layout_passes", False),
                   ("use_tc_tiling_on_sc", False)):
        if _f in pltpu.CompilerParams.__dataclass_fields__:
            cp = dataclasses.replace(cp, **{_f: _v})

    mesh = plsc.VectorSubcoreMesh(core_axis_name="c", subcore_axis_name="s")
    run = pl.kernel(
        _encoder_body,
        out_type=jax.ShapeDtypeStruct((n * OUT_F,), jnp.float32),
        mesh=mesh,
        compiler_params=cp,
        scratch_types=[
            pltpu.VMEM((CHUNK,), jnp.float32),
            pltpu.VMEM((CHUNK,), jnp.float32),
            pltpu.VMEM((CHUNK,), jnp.float32),
            pltpu.VMEM((DEPTH, 8, CHUNK), jnp.int32),
            pltpu.VMEM((DEPTH, 8, CHUNK), jnp.float32),
            pltpu.VMEM((DEPTH, 8, CHUNK), jnp.float32),
            pltpu.VMEM((DEPTH, 8, CHUNK), jnp.float32),
            pltpu.VMEM((CHUNK * OUT_F,), jnp.float32),
            pltpu.VMEM((_OFFSETS[1] - _OFFSETS[0],), jnp.float32),
            pltpu.VMEM((_OFFSETS[1] - _OFFSETS[0],), jnp.float32),
            pltpu.VMEM((_OFFSETS[2] - _OFFSETS[1],), jnp.float32),
            pltpu.VMEM((_OFFSETS[2] - _OFFSETS[1],), jnp.float32),
            pltpu.VMEM((_OFFSETS[3] - _OFFSETS[2],), jnp.float32),
            pltpu.VMEM((_OFFSETS[3] - _OFFSETS[2],), jnp.float32),
            pltpu.SemaphoreType.DMA,
        ],
    )
    e0 = embeddings[:, 0]
    e1 = embeddings[:, 1]
    out = run(x, y, z, e0, e1)
    return out.reshape(n, OUT_F)
